# baseline (speedup 1.0000x reference)
import functools

import jax
import jax.numpy as jnp
from jax.experimental import pallas as pl
from jax.experimental.pallas import tpu as pltpu

_NUM_EMBEDDING = 1024
_EMBEDDING_DIM = 64
_COMMITMENT_COST = 0.25
_BT = 1024


def _vq_block(x_ref, xsq_ref, esq_ref, emb_ref, emb2_ref,
              enc_ref, q_ref, counts_ref, sumsq_ref, loss_ref, perp_ref):
    b = pl.program_id(0)
    nb = pl.num_programs(0)

    x = x_ref[0]
    emb = emb_ref[...]

    mm2 = jax.lax.dot_general(
        x, emb2_ref[...], (((0,), (1,)), ((), ())),
        preferred_element_type=jnp.float32)
    d = (xsq_ref[...] + esq_ref[...]) - mm2

    lanes = jax.lax.broadcasted_iota(jnp.int32, (_BT, _NUM_EMBEDDING), 1)
    dmin = jnp.min(d, axis=1, keepdims=True)
    idx = jnp.min(jnp.where(d == dmin, lanes, _NUM_EMBEDDING),
                  axis=1, keepdims=True)
    oh = (lanes == idx).astype(jnp.float32)
    enc_ref[...] = oh

    q = jax.lax.dot_general(
        emb, oh, (((0,), (1,)), ((), ())),
        preferred_element_type=jnp.float32)
    q_ref[0] = q

    diff = q - x
    part_sumsq = jnp.sum(diff * diff, axis=(0, 1), keepdims=True)
    part_counts = jnp.sum(oh, axis=0, keepdims=True)

    @pl.when(b == 0)
    def _():
        sumsq_ref[...] = part_sumsq
        counts_ref[...] = part_counts

    @pl.when(b > 0)
    def _():
        sumsq_ref[...] += part_sumsq
        counts_ref[...] += part_counts

    @pl.when(b == nb - 1)
    def _():
        n_tok = jnp.float32(nb * _BT)
        mean_sq = sumsq_ref[...] / (n_tok * jnp.float32(_EMBEDDING_DIM))
        loss_ref[...] = (1.0 + _COMMITMENT_COST) * mean_sq
        probs = counts_ref[...] / n_tok
        ent = jnp.sum(probs * jnp.log(probs + 1e-10),
                      axis=(0, 1), keepdims=True)
        perp_ref[...] = jnp.exp(-ent)


@functools.partial(jax.jit, static_argnums=())
def kernel(inputs, embedding):
    bsz, chan, hh, ww = inputs.shape
    n = bsz * hh * ww
    grid = n // _BT

    x = jnp.transpose(inputs, (0, 2, 3, 1))
    flat_x = x.reshape(-1, _EMBEDDING_DIM)
    xsq = jnp.sum(flat_x ** 2, axis=1, keepdims=True)
    esq = jnp.sum(embedding ** 2, axis=1).reshape(1, -1)
    emb2 = embedding + embedding

    x_cm = inputs.reshape(bsz, chan, hh * ww)

    enc, q, _counts, _sumsq, loss, perp = pl.pallas_call(
        _vq_block,
        grid=(grid,),
        in_specs=[
            pl.BlockSpec((1, chan, _BT), lambda b: (b, 0, 0)),
            pl.BlockSpec((_BT, 1), lambda b: (b, 0)),
            pl.BlockSpec((1, _NUM_EMBEDDING), lambda b: (0, 0)),
            pl.BlockSpec((_NUM_EMBEDDING, _EMBEDDING_DIM), lambda b: (0, 0)),
            pl.BlockSpec((_NUM_EMBEDDING, _EMBEDDING_DIM), lambda b: (0, 0)),
        ],
        out_specs=[
            pl.BlockSpec((_BT, _NUM_EMBEDDING), lambda b: (b, 0)),
            pl.BlockSpec((1, chan, _BT), lambda b: (b, 0, 0)),
            pl.BlockSpec((1, _NUM_EMBEDDING), lambda b: (0, 0)),
            pl.BlockSpec((1, 1), lambda b: (0, 0)),
            pl.BlockSpec((1, 1), lambda b: (0, 0)),
            pl.BlockSpec((1, 1), lambda b: (0, 0)),
        ],
        out_shape=[
            jax.ShapeDtypeStruct((n, _NUM_EMBEDDING), jnp.float32),
            jax.ShapeDtypeStruct((bsz, chan, hh * ww), jnp.float32),
            jax.ShapeDtypeStruct((1, _NUM_EMBEDDING), jnp.float32),
            jax.ShapeDtypeStruct((1, 1), jnp.float32),
            jax.ShapeDtypeStruct((1, 1), jnp.float32),
            jax.ShapeDtypeStruct((1, 1), jnp.float32),
        ],
        compiler_params=pltpu.CompilerParams(
            dimension_semantics=("arbitrary",)),
    )(x_cm, xsq, esq, embedding, emb2)

    quantized_convert = q.reshape(bsz, chan, hh, ww)
    return (loss[0, 0], quantized_convert, perp[0, 0], enc)


# speedup vs baseline: 1.0022x; 1.0022x over previous
import functools

import jax
import jax.numpy as jnp
from jax.experimental import pallas as pl
from jax.experimental.pallas import tpu as pltpu

_NUM_EMBEDDING = 1024
_EMBEDDING_DIM = 64
_COMMITMENT_COST = 0.25
_BT = 1024


def _vq_block(x_ref, xsq_ref, esq_ref, emb_ref, emb2_ref,
              enc_ref, q_ref, loss_ref, perp_ref,
              counts_ref, sumsq_ref):
    b = pl.program_id(0)
    nb = pl.num_programs(0)

    x = x_ref[0]
    emb = emb_ref[...]

    mm2 = jax.lax.dot_general(
        x, emb2_ref[...], (((0,), (1,)), ((), ())),
        preferred_element_type=jnp.float32)
    d = (xsq_ref[...] + esq_ref[...]) - mm2

    lanes = jax.lax.broadcasted_iota(jnp.int32, (_BT, _NUM_EMBEDDING), 1)
    dmin = jnp.min(d, axis=1, keepdims=True)
    idx = jnp.min(jnp.where(d == dmin, lanes, _NUM_EMBEDDING),
                  axis=1, keepdims=True)
    oh = (lanes == idx).astype(jnp.float32)
    enc_ref[...] = oh

    q = jax.lax.dot_general(
        emb, oh, (((0,), (1,)), ((), ())),
        preferred_element_type=jnp.float32)
    q_ref[0] = q

    diff = q - x
    part_sumsq = jnp.sum(diff * diff, axis=(0, 1), keepdims=True)
    part_counts = jnp.sum(oh, axis=0, keepdims=True)

    @pl.when(b == 0)
    def _():
        sumsq_ref[...] = part_sumsq
        counts_ref[...] = part_counts

    @pl.when(b > 0)
    def _():
        sumsq_ref[...] += part_sumsq
        counts_ref[...] += part_counts

    @pl.when(b == nb - 1)
    def _():
        n_tok = jnp.float32(nb * _BT)
        mean_sq = sumsq_ref[...] / (n_tok * jnp.float32(_EMBEDDING_DIM))
        loss_ref[...] = (1.0 + _COMMITMENT_COST) * mean_sq
        probs = counts_ref[...] / n_tok
        ent = jnp.sum(probs * jnp.log(probs + 1e-10),
                      axis=(0, 1), keepdims=True)
        perp_ref[...] = jnp.exp(-ent)


@functools.partial(jax.jit, static_argnums=())
def kernel(inputs, embedding):
    bsz, chan, hh, ww = inputs.shape
    n = bsz * hh * ww
    grid = n // _BT

    x = jnp.transpose(inputs, (0, 2, 3, 1))
    flat_x = x.reshape(-1, _EMBEDDING_DIM)
    xsq = jnp.sum(flat_x ** 2, axis=1, keepdims=True)
    esq = jnp.sum(embedding ** 2, axis=1).reshape(1, -1)
    emb2 = embedding + embedding

    x_cm = inputs.reshape(bsz, chan, hh * ww)

    enc, q, loss, perp = pl.pallas_call(
        _vq_block,
        grid=(grid,),
        in_specs=[
            pl.BlockSpec((1, chan, _BT), lambda b: (b, 0, 0)),
            pl.BlockSpec((_BT, 1), lambda b: (b, 0)),
            pl.BlockSpec((1, _NUM_EMBEDDING), lambda b: (0, 0)),
            pl.BlockSpec((_NUM_EMBEDDING, _EMBEDDING_DIM), lambda b: (0, 0)),
            pl.BlockSpec((_NUM_EMBEDDING, _EMBEDDING_DIM), lambda b: (0, 0)),
        ],
        out_specs=[
            pl.BlockSpec((_BT, _NUM_EMBEDDING), lambda b: (b, 0)),
            pl.BlockSpec((1, chan, _BT), lambda b: (b, 0, 0)),
            pl.BlockSpec((1, 1), lambda b: (0, 0)),
            pl.BlockSpec((1, 1), lambda b: (0, 0)),
        ],
        out_shape=[
            jax.ShapeDtypeStruct((n, _NUM_EMBEDDING), jnp.float32),
            jax.ShapeDtypeStruct((bsz, chan, hh * ww), jnp.float32),
            jax.ShapeDtypeStruct((1, 1), jnp.float32),
            jax.ShapeDtypeStruct((1, 1), jnp.float32),
        ],
        scratch_shapes=[
            pltpu.VMEM((1, _NUM_EMBEDDING), jnp.float32),
            pltpu.VMEM((1, 1), jnp.float32),
        ],
        compiler_params=pltpu.CompilerParams(
            dimension_semantics=("arbitrary",)),
    )(x_cm, xsq, esq, embedding, emb2)

    quantized_convert = q.reshape(bsz, chan, hh, ww)
    return (loss[0, 0], quantized_convert, perp[0, 0], enc)


# speedup vs baseline: 1.1966x; 1.1940x over previous
import functools

import jax
import jax.numpy as jnp
from jax.experimental import pallas as pl
from jax.experimental.pallas import tpu as pltpu

_NUM_EMBEDDING = 1024
_EMBEDDING_DIM = 64
_COMMITMENT_COST = 0.25
_BT = 1024


def _vq_block(x_ref, esq_ref, emb_ref, emb2_ref,
              enc_ref, q_ref, loss_ref, perp_ref,
              counts_ref, sumsq_ref):
    b = pl.program_id(0)
    nb = pl.num_programs(0)

    x = x_ref[0]
    emb = emb_ref[...]

    mm2 = jax.lax.dot_general(
        x, emb2_ref[...], (((0,), (1,)), ((), ())),
        preferred_element_type=jnp.float32)
    xsq = jnp.sum(x * x, axis=0).reshape(_BT, 1)
    d = (xsq + esq_ref[...]) - mm2

    lanes = jax.lax.broadcasted_iota(jnp.int32, (_BT, _NUM_EMBEDDING), 1)
    dmin = jnp.min(d, axis=1, keepdims=True)
    idx = jnp.min(jnp.where(d == dmin, lanes, _NUM_EMBEDDING),
                  axis=1, keepdims=True)
    oh = (lanes == idx).astype(jnp.float32)
    enc_ref[...] = oh

    q = jax.lax.dot_general(
        emb, oh, (((0,), (1,)), ((), ())),
        preferred_element_type=jnp.float32)
    q_ref[0] = q

    diff = q - x
    part_sumsq = jnp.sum(diff * diff, axis=(0, 1), keepdims=True)
    part_counts = jnp.sum(oh, axis=0, keepdims=True)

    @pl.when(b == 0)
    def _():
        sumsq_ref[...] = part_sumsq
        counts_ref[...] = part_counts

    @pl.when(b > 0)
    def _():
        sumsq_ref[...] += part_sumsq
        counts_ref[...] += part_counts

    @pl.when(b == nb - 1)
    def _():
        n_tok = jnp.float32(nb * _BT)
        mean_sq = sumsq_ref[...] / (n_tok * jnp.float32(_EMBEDDING_DIM))
        loss_ref[...] = (1.0 + _COMMITMENT_COST) * mean_sq
        probs = counts_ref[...] / n_tok
        ent = jnp.sum(probs * jnp.log(probs + 1e-10),
                      axis=(0, 1), keepdims=True)
        perp_ref[...] = jnp.exp(-ent)


@functools.partial(jax.jit, static_argnums=())
def kernel(inputs, embedding):
    bsz, chan, hh, ww = inputs.shape
    n = bsz * hh * ww
    grid = n // _BT

    esq = jnp.sum(embedding ** 2, axis=1).reshape(1, -1)
    emb2 = embedding + embedding

    x_cm = inputs.reshape(bsz, chan, hh * ww)

    enc, q, loss, perp = pl.pallas_call(
        _vq_block,
        grid=(grid,),
        in_specs=[
            pl.BlockSpec((1, chan, _BT), lambda b: (b, 0, 0)),
            pl.BlockSpec((1, _NUM_EMBEDDING), lambda b: (0, 0)),
            pl.BlockSpec((_NUM_EMBEDDING, _EMBEDDING_DIM), lambda b: (0, 0)),
            pl.BlockSpec((_NUM_EMBEDDING, _EMBEDDING_DIM), lambda b: (0, 0)),
        ],
        out_specs=[
            pl.BlockSpec((_BT, _NUM_EMBEDDING), lambda b: (b, 0)),
            pl.BlockSpec((1, chan, _BT), lambda b: (b, 0, 0)),
            pl.BlockSpec((1, 1), lambda b: (0, 0)),
            pl.BlockSpec((1, 1), lambda b: (0, 0)),
        ],
        out_shape=[
            jax.ShapeDtypeStruct((n, _NUM_EMBEDDING), jnp.float32),
            jax.ShapeDtypeStruct((bsz, chan, hh * ww), jnp.float32),
            jax.ShapeDtypeStruct((1, 1), jnp.float32),
            jax.ShapeDtypeStruct((1, 1), jnp.float32),
        ],
        scratch_shapes=[
            pltpu.VMEM((1, _NUM_EMBEDDING), jnp.float32),
            pltpu.VMEM((1, 1), jnp.float32),
        ],
        compiler_params=pltpu.CompilerParams(
            dimension_semantics=("arbitrary",)),
    )(x_cm, esq, embedding, emb2)

    quantized_convert = q.reshape(bsz, chan, hh, ww)
    return (loss[0, 0], quantized_convert, perp[0, 0], enc)
